# baseline (speedup 1.0000x reference)
import jax
import jax.numpy as jnp
from jax import lax
from jax.experimental import pallas as pl
from jax.experimental.pallas import tpu as pltpu
from jax.experimental.pallas import tpu_sc as plsc

B, IN, HID, OUT = 256, 16384, 4096, 64
NNZ1, NNZ2 = 131072, 16384
NC, NS = 2, 16
HB = B // NC
CH = 128
G1 = NNZ1 // NS // CH
G2 = NNZ2 // NS // CH
R1 = HID // NS
NV = HB // 16


def _scale_chunk(rows_v, wb, g):
    @plsc.parallel_loop(0, CH // 16)
    def grp(t):
        wvec = wb[g, pl.ds(t * 16, 16)]
        for i in range(16):
            bv = jnp.full((16,), wvec[i], jnp.float32)
            j = t * 16 + i
            for k in range(NV):
                rows_v[j, pl.ds(k * 16, 16)] = rows_v[j, pl.ds(k * 16, 16)] * bv


NB = 4


def _layer(table, idxb, oidxb, wb, nchunks, acc_sh, rows, gsems, ssems):
    for p in range(NB):
        pltpu.async_copy(table.at[idxb.at[p]], rows[p], gsems[p])

    def outer(m, _):
        for p in range(NB):
            g = NB * m + p
            pltpu.make_async_copy(table.at[idxb.at[g]], rows[p], gsems[p]).wait()
            _scale_chunk(rows[p], wb, g)
            pltpu.async_copy(rows[p], acc_sh.at[oidxb.at[g]], ssems[p], add=True)
            q = (p + 2) % NB
            gq = g - 2

            @pl.when(jnp.logical_and(gq >= 0, gq + NB < nchunks))
            def _():
                pltpu.make_async_copy(rows[q], acc_sh.at[oidxb.at[gq]],
                                      ssems[q]).wait()
                pltpu.async_copy(table.at[idxb.at[gq + NB]], rows[q], gsems[q])
        return 0
    lax.fori_loop(0, nchunks // NB, outer, 0)

    for p in range(NB):
        g = nchunks - NB + p
        pltpu.make_async_copy(rows[p], acc_sh.at[oidxb.at[g]], ssems[p]).wait()


def _add_offset(idxb, nrows, off):
    def offrow(r, _):
        for k in range(NV):
            idxb[r, pl.ds(k * 16, 16)] = idxb[r, pl.ds(k * 16, 16)] + off
        return 0
    lax.fori_loop(0, nrows, offrow, 0)


def _sc_body(x2, in1, out1, w1, b1, in2, out2, w2, b2,
             h_hbm, o_hbm,
             acc_sh, acc2_sh, idxb, oidxb, wb,
             rows0, rows1, rows2, rows3, bbuf, b2buf,
             gs0, gs1, gs2, gs3, ss0, ss1, ss2, ss3):
    rows = [rows0, rows1, rows2, rows3]
    sbuf = rows0
    gsems = [gs0, gs1, gs2, gs3]
    ssems = [ss0, ss1, ss2, ss3]
    c = lax.axis_index("c")
    s = lax.axis_index("s")

    pltpu.sync_copy(b1.at[pl.ds(s * R1, R1)], bbuf.at[pl.ds(0, R1)])

    def init_blk(blk, _):
        def init_row(r, _):
            bv = jnp.full((16,), bbuf[pl.ds(blk * 64 + r, 16)][0], jnp.float32)
            for k in range(NV):
                sbuf[r, pl.ds(k * 16, 16)] = bv
            return 0
        lax.fori_loop(0, 64, init_row, 0)
        pltpu.sync_copy(sbuf.at[pl.ds(0, 64)],
                        acc_sh.at[pl.ds(s * R1 + blk * 64, 64)])
        return 0
    lax.fori_loop(0, R1 // 64, init_blk, 0)

    @pl.when(s < 8)
    def _():
        pltpu.sync_copy(b2.at[pl.ds(s * 8, 8)], b2buf.at[pl.ds(0, 8)])

        def init2_row(r, _):
            bv = jnp.full((16,), b2buf[pl.ds(r, 16)][0], jnp.float32)
            for k in range(NV):
                sbuf[r, pl.ds(k * 16, 16)] = bv
            return 0
        lax.fori_loop(0, 8, init2_row, 0)
        pltpu.sync_copy(sbuf.at[pl.ds(0, 8)], acc2_sh.at[pl.ds(s * 8, 8)])

    pltpu.sync_copy(in1.at[pl.ds(s * G1, G1)], idxb.at[pl.ds(0, G1)])
    pltpu.sync_copy(out1.at[pl.ds(s * G1, G1)], oidxb.at[pl.ds(0, G1)])
    pltpu.sync_copy(w1.at[pl.ds(s * G1, G1)], wb.at[pl.ds(0, G1)])
    _add_offset(idxb, G1, c * IN)

    plsc.subcore_barrier()

    _layer(x2, idxb, oidxb, wb, G1, acc_sh, rows, gsems, ssems)

    plsc.subcore_barrier()

    def sig_blk(blk, _):
        r0 = s * R1 + blk * 64
        pltpu.sync_copy(acc_sh.at[pl.ds(r0, 64)], sbuf.at[pl.ds(0, 64)])

        def sig_row(r, _):
            for k in range(NV):
                z = sbuf[r, pl.ds(k * 16, 16)]
                sbuf[r, pl.ds(k * 16, 16)] = 1.0 / (1.0 + jnp.exp(-z))
            return 0
        lax.fori_loop(0, 64, sig_row, 0)
        pltpu.sync_copy(sbuf.at[pl.ds(0, 64)],
                        h_hbm.at[pl.ds(c * HID + r0, 64)])
        return 0
    lax.fori_loop(0, R1 // 64, sig_blk, 0)

    pltpu.sync_copy(in2.at[pl.ds(s * G2, G2)], idxb.at[pl.ds(0, G2)])
    pltpu.sync_copy(out2.at[pl.ds(s * G2, G2)], oidxb.at[pl.ds(0, G2)])
    pltpu.sync_copy(w2.at[pl.ds(s * G2, G2)], wb.at[pl.ds(0, G2)])
    _add_offset(idxb, G2, c * HID)

    plsc.subcore_barrier()

    _layer(h_hbm, idxb, oidxb, wb, G2, acc2_sh, rows, gsems, ssems)

    plsc.subcore_barrier()

    @pl.when(s < 8)
    def _():
        pltpu.sync_copy(acc2_sh.at[pl.ds(s * 8, 8)], o_hbm.at[c, pl.ds(s * 8, 8)])


def kernel(x, connections1, connections2, w1, b1, w2, b2):
    x2 = x.reshape(NC, HB, IN).transpose(0, 2, 1).reshape(NC * IN, HB)
    in1 = connections1[1].reshape(NNZ1 // CH, CH)
    out1 = connections1[0].reshape(NNZ1 // CH, CH)
    in2 = connections2[1].reshape(NNZ2 // CH, CH)
    out2 = connections2[0].reshape(NNZ2 // CH, CH)
    w1r = w1.reshape(NNZ1 // CH, CH)
    w2r = w2.reshape(NNZ2 // CH, CH)

    mesh = plsc.VectorSubcoreMesh(core_axis_name="c", subcore_axis_name="s")
    h_hbm, o_hbm = pl.kernel(
        _sc_body,
        out_type=(
            jax.ShapeDtypeStruct((NC * HID, HB), jnp.float32),
            jax.ShapeDtypeStruct((NC, OUT, HB), jnp.float32),
        ),
        mesh=mesh,
        scratch_types=(
            pltpu.VMEM_SHARED((HID, HB), jnp.float32),
            pltpu.VMEM_SHARED((OUT, HB), jnp.float32),
            pltpu.VMEM((G1, CH), jnp.int32),
            pltpu.VMEM((G1, CH), jnp.int32),
            pltpu.VMEM((G1, CH), jnp.float32),
            pltpu.VMEM((CH, HB), jnp.float32),
            pltpu.VMEM((CH, HB), jnp.float32),
            pltpu.VMEM((CH, HB), jnp.float32),
            pltpu.VMEM((CH, HB), jnp.float32),
            pltpu.VMEM((R1 + 16,), jnp.float32),
            pltpu.VMEM((24,), jnp.float32),
            pltpu.SemaphoreType.DMA,
            pltpu.SemaphoreType.DMA,
            pltpu.SemaphoreType.DMA,
            pltpu.SemaphoreType.DMA,
            pltpu.SemaphoreType.DMA,
            pltpu.SemaphoreType.DMA,
            pltpu.SemaphoreType.DMA,
            pltpu.SemaphoreType.DMA,
        ),
    )(x2, in1, out1, w1r, b1, in2, out2, w2r, b2)
    del h_hbm
    return o_hbm.transpose(0, 2, 1).reshape(B, OUT)


# speedup vs baseline: 1.0661x; 1.0661x over previous
import jax
import jax.numpy as jnp
from jax import lax
from jax.experimental import pallas as pl
from jax.experimental.pallas import tpu as pltpu
from jax.experimental.pallas import tpu_sc as plsc

B, IN, HID, OUT = 256, 16384, 4096, 64
NNZ1, NNZ2 = 131072, 16384
NC, NS = 2, 16
HB = B // NC
CH = 128
G1 = NNZ1 // NS // CH
G2 = NNZ2 // NS // CH
R1 = HID // NS
NV = HB // 16
NB = 4


def _scale_chunk(rows_v, wb, g):
    def grp(t, _):
        wvec = wb[g, pl.ds(t * 16, 16)]
        for i in range(16):
            bv = jnp.full((16,), wvec[i], jnp.float32)
            j = t * 16 + i
            for k in range(NV):
                rows_v[j, pl.ds(k * 16, 16)] = rows_v[j, pl.ds(k * 16, 16)] * bv
        return 0
    lax.fori_loop(0, CH // 16, grp, 0)


def _layer(table, idxb, oidxb, wb, nchunks, acc_sh, rows, gsems, ssems):
    for p in range(NB):
        pltpu.async_copy(table.at[idxb.at[p]], rows[p], gsems[p])

    def outer(m, _):
        for p in range(NB):
            g = NB * m + p
            q = (p + NB // 2) % NB
            gq = g - NB // 2

            @pl.when(jnp.logical_and(gq >= 0, gq + NB < nchunks))
            def _():
                pltpu.make_async_copy(rows[q], acc_sh.at[oidxb.at[gq]],
                                      ssems[q]).wait()
                pltpu.async_copy(table.at[idxb.at[gq + NB]], rows[q], gsems[q])

            pltpu.make_async_copy(table.at[idxb.at[g]], rows[p], gsems[p]).wait()
            _scale_chunk(rows[p], wb, g)
            pltpu.async_copy(rows[p], acc_sh.at[oidxb.at[g]], ssems[p], add=True)
        return 0
    lax.fori_loop(0, nchunks // NB, outer, 0)

    for p in range(NB):
        g = nchunks - NB + p
        pltpu.make_async_copy(rows[p], acc_sh.at[oidxb.at[g]], ssems[p]).wait()


def _add_offset(idxb, nrows, off):
    def offrow(r, _):
        for k in range(NV):
            idxb[r, pl.ds(k * 16, 16)] = idxb[r, pl.ds(k * 16, 16)] + off
        return 0
    lax.fori_loop(0, nrows, offrow, 0)


def _sc_body(x2, in1, out1, w1, b1, in2, out2, w2, b2,
             h_hbm, o_hbm,
             acc_sh, acc2_sh, idxb, oidxb, wb,
             rows0, rows1, rows2, rows3, bbuf, b2buf,
             gs0, gs1, gs2, gs3, ss0, ss1, ss2, ss3):
    rows = [rows0, rows1, rows2, rows3]
    gsems = [gs0, gs1, gs2, gs3]
    ssems = [ss0, ss1, ss2, ss3]
    sbuf = rows0
    c = lax.axis_index("c")
    s = lax.axis_index("s")

    pltpu.sync_copy(b1.at[pl.ds(s * R1, R1)], bbuf.at[pl.ds(0, R1)])

    def init_blk(blk, _):
        def init_row(r, _):
            bv = jnp.full((16,), bbuf[pl.ds(blk * 64 + r, 16)][0], jnp.float32)
            for k in range(NV):
                sbuf[r, pl.ds(k * 16, 16)] = bv
            return 0
        lax.fori_loop(0, 64, init_row, 0)
        pltpu.sync_copy(sbuf.at[pl.ds(0, 64)],
                        acc_sh.at[pl.ds(s * R1 + blk * 64, 64)])
        return 0
    lax.fori_loop(0, R1 // 64, init_blk, 0)

    @pl.when(s < 8)
    def _():
        pltpu.sync_copy(b2.at[pl.ds(s * 8, 8)], b2buf.at[pl.ds(0, 8)])

        def init2_row(r, _):
            bv = jnp.full((16,), b2buf[pl.ds(r, 16)][0], jnp.float32)
            for k in range(NV):
                sbuf[r, pl.ds(k * 16, 16)] = bv
            return 0
        lax.fori_loop(0, 8, init2_row, 0)
        pltpu.sync_copy(sbuf.at[pl.ds(0, 8)], acc2_sh.at[pl.ds(s * 8, 8)])

    pltpu.sync_copy(in1.at[pl.ds(s * G1, G1)], idxb.at[pl.ds(0, G1)])
    pltpu.sync_copy(out1.at[pl.ds(s * G1, G1)], oidxb.at[pl.ds(0, G1)])
    pltpu.sync_copy(w1.at[pl.ds(s * G1, G1)], wb.at[pl.ds(0, G1)])
    _add_offset(idxb, G1, c * IN)

    plsc.subcore_barrier()

    _layer(x2, idxb, oidxb, wb, G1, acc_sh, rows, gsems, ssems)

    plsc.subcore_barrier()

    def sig_blk(blk, _):
        r0 = s * R1 + blk * 64
        pltpu.sync_copy(acc_sh.at[pl.ds(r0, 64)], sbuf.at[pl.ds(0, 64)])

        def sig_row(r, _):
            for k in range(NV):
                z = sbuf[r, pl.ds(k * 16, 16)]
                sbuf[r, pl.ds(k * 16, 16)] = 1.0 / (1.0 + jnp.exp(-z))
            return 0
        lax.fori_loop(0, 64, sig_row, 0)
        pltpu.sync_copy(sbuf.at[pl.ds(0, 64)],
                        h_hbm.at[pl.ds(c * HID + r0, 64)])
        return 0
    lax.fori_loop(0, R1 // 64, sig_blk, 0)

    pltpu.sync_copy(in2.at[pl.ds(s * G2, G2)], idxb.at[pl.ds(0, G2)])
    pltpu.sync_copy(out2.at[pl.ds(s * G2, G2)], oidxb.at[pl.ds(0, G2)])
    pltpu.sync_copy(w2.at[pl.ds(s * G2, G2)], wb.at[pl.ds(0, G2)])
    _add_offset(idxb, G2, c * HID)

    plsc.subcore_barrier()

    _layer(h_hbm, idxb, oidxb, wb, G2, acc2_sh, rows, gsems, ssems)

    plsc.subcore_barrier()

    @pl.when(s < 8)
    def _():
        pltpu.sync_copy(acc2_sh.at[pl.ds(s * 8, 8)], o_hbm.at[c, pl.ds(s * 8, 8)])


def kernel(x, connections1, connections2, w1, b1, w2, b2):
    x2 = x.reshape(NC, HB, IN).transpose(0, 2, 1).reshape(NC * IN, HB)
    in1 = connections1[1].reshape(NNZ1 // CH, CH)
    out1 = connections1[0].reshape(NNZ1 // CH, CH)
    in2 = connections2[1].reshape(NNZ2 // CH, CH)
    out2 = connections2[0].reshape(NNZ2 // CH, CH)
    w1r = w1.reshape(NNZ1 // CH, CH)
    w2r = w2.reshape(NNZ2 // CH, CH)

    mesh = plsc.VectorSubcoreMesh(core_axis_name="c", subcore_axis_name="s")
    h_hbm, o_hbm = pl.kernel(
        _sc_body,
        out_type=(
            jax.ShapeDtypeStruct((NC * HID, HB), jnp.float32),
            jax.ShapeDtypeStruct((NC, OUT, HB), jnp.float32),
        ),
        mesh=mesh,
        scratch_types=(
            pltpu.VMEM_SHARED((HID, HB), jnp.float32),
            pltpu.VMEM_SHARED((OUT, HB), jnp.float32),
            pltpu.VMEM((G1, CH), jnp.int32),
            pltpu.VMEM((G1, CH), jnp.int32),
            pltpu.VMEM((G1, CH), jnp.float32),
            pltpu.VMEM((CH, HB), jnp.float32),
            pltpu.VMEM((CH, HB), jnp.float32),
            pltpu.VMEM((CH, HB), jnp.float32),
            pltpu.VMEM((CH, HB), jnp.float32),
            pltpu.VMEM((R1 + 16,), jnp.float32),
            pltpu.VMEM((24,), jnp.float32),
            pltpu.SemaphoreType.DMA,
            pltpu.SemaphoreType.DMA,
            pltpu.SemaphoreType.DMA,
            pltpu.SemaphoreType.DMA,
            pltpu.SemaphoreType.DMA,
            pltpu.SemaphoreType.DMA,
            pltpu.SemaphoreType.DMA,
            pltpu.SemaphoreType.DMA,
        ),
    )(x2, in1, out1, w1r, b1, in2, out2, w2r, b2)
    del h_hbm
    return o_hbm.transpose(0, 2, 1).reshape(B, OUT)


# speedup vs baseline: 1.0833x; 1.0162x over previous
import jax
import jax.numpy as jnp
from jax import lax
from jax.experimental import pallas as pl
from jax.experimental.pallas import tpu as pltpu
from jax.experimental.pallas import tpu_sc as plsc

B, IN, HID, OUT = 256, 16384, 4096, 64
NNZ1, NNZ2 = 131072, 16384
NC, NS = 2, 16
HB = B // NC
CH = 128
G1 = NNZ1 // NS // CH
G2 = NNZ2 // NS // CH
R1 = HID // NS
NV = HB // 16
NB = 4


def _scale_chunk(rows_v, wb, g):
    def grp(t, _):
        wvec = wb[g, pl.ds(t * 16, 16)]
        for i in range(16):
            bv = jnp.full((16,), wvec[i], jnp.float32)
            j = t * 16 + i
            for k in range(NV):
                rows_v[j, pl.ds(k * 16, 16)] = rows_v[j, pl.ds(k * 16, 16)] * bv
        return 0
    lax.fori_loop(0, CH // 16, grp, 0)


def _layer(table, idxb, oidxb, wb, nchunks, acc_sh, rows, gsems, ssems):
    for p in range(NB):
        pltpu.async_copy(table.at[idxb.at[p]], rows[p], gsems[p])

    def outer(m, _):
        for p in range(NB):
            g = NB * m + p
            pltpu.make_async_copy(table.at[idxb.at[g]], rows[p], gsems[p]).wait()
            _scale_chunk(rows[p], wb, g)
            pltpu.async_copy(rows[p], acc_sh.at[oidxb.at[g]], ssems[p], add=True)
            q = (p + NB // 2) % NB
            gq = g - NB // 2

            @pl.when(jnp.logical_and(gq >= 0, gq + NB < nchunks))
            def _():
                pltpu.make_async_copy(rows[q], acc_sh.at[oidxb.at[gq]],
                                      ssems[q]).wait()
                pltpu.async_copy(table.at[idxb.at[gq + NB]], rows[q], gsems[q])
        return 0
    lax.fori_loop(0, nchunks // NB, outer, 0)

    for p in range(NB):
        g = nchunks - NB + p
        pltpu.make_async_copy(rows[p], acc_sh.at[oidxb.at[g]], ssems[p]).wait()


def _add_offset(idxb, nrows, off):
    def offrow(r, _):
        for k in range(NV):
            idxb[r, pl.ds(k * 16, 16)] = idxb[r, pl.ds(k * 16, 16)] + off
        return 0
    lax.fori_loop(0, nrows, offrow, 0)


def _sc_body(x2, in1, out1, w1, b1, in2, out2, w2, b2,
             h_hbm, o_hbm,
             acc_sh, acc2_sh, idxb, oidxb, wb,
             rows0, rows1, rows2, rows3, bbuf, b2buf,
             gs0, gs1, gs2, gs3, ss0, ss1, ss2, ss3):
    rows = [rows0, rows1, rows2, rows3]
    gsems = [gs0, gs1, gs2, gs3]
    ssems = [ss0, ss1, ss2, ss3]
    sbuf = rows0
    c = lax.axis_index("c")
    s = lax.axis_index("s")

    pltpu.sync_copy(b1.at[pl.ds(s * R1, R1)], bbuf.at[pl.ds(0, R1)])

    def init_blk(blk, _):
        def init_row(r, _):
            bv = jnp.full((16,), bbuf[pl.ds(blk * 64 + r, 16)][0], jnp.float32)
            for k in range(NV):
                sbuf[r, pl.ds(k * 16, 16)] = bv
            return 0
        lax.fori_loop(0, 64, init_row, 0)
        pltpu.sync_copy(sbuf.at[pl.ds(0, 64)],
                        acc_sh.at[pl.ds(s * R1 + blk * 64, 64)])
        return 0
    lax.fori_loop(0, R1 // 64, init_blk, 0)

    @pl.when(s < 8)
    def _():
        pltpu.sync_copy(b2.at[pl.ds(s * 8, 8)], b2buf.at[pl.ds(0, 8)])

        def init2_row(r, _):
            bv = jnp.full((16,), b2buf[pl.ds(r, 16)][0], jnp.float32)
            for k in range(NV):
                sbuf[r, pl.ds(k * 16, 16)] = bv
            return 0
        lax.fori_loop(0, 8, init2_row, 0)
        pltpu.sync_copy(sbuf.at[pl.ds(0, 8)], acc2_sh.at[pl.ds(s * 8, 8)])

    pltpu.sync_copy(in1.at[pl.ds(s * G1, G1)], idxb.at[pl.ds(0, G1)])
    pltpu.sync_copy(out1.at[pl.ds(s * G1, G1)], oidxb.at[pl.ds(0, G1)])
    pltpu.sync_copy(w1.at[pl.ds(s * G1, G1)], wb.at[pl.ds(0, G1)])
    _add_offset(idxb, G1, c * IN)

    plsc.subcore_barrier()

    _layer(x2, idxb, oidxb, wb, G1, acc_sh, rows, gsems, ssems)

    plsc.subcore_barrier()

    def sig_blk(blk, _):
        r0 = s * R1 + blk * 64
        pltpu.sync_copy(acc_sh.at[pl.ds(r0, 64)], sbuf.at[pl.ds(0, 64)])

        def sig_row(r, _):
            for k in range(NV):
                z = sbuf[r, pl.ds(k * 16, 16)]
                sbuf[r, pl.ds(k * 16, 16)] = 1.0 / (1.0 + jnp.exp(-z))
            return 0
        lax.fori_loop(0, 64, sig_row, 0)
        pltpu.sync_copy(sbuf.at[pl.ds(0, 64)],
                        h_hbm.at[pl.ds(c * HID + r0, 64)])
        return 0
    lax.fori_loop(0, R1 // 64, sig_blk, 0)

    pltpu.sync_copy(in2.at[pl.ds(s * G2, G2)], idxb.at[pl.ds(0, G2)])
    pltpu.sync_copy(out2.at[pl.ds(s * G2, G2)], oidxb.at[pl.ds(0, G2)])
    pltpu.sync_copy(w2.at[pl.ds(s * G2, G2)], wb.at[pl.ds(0, G2)])
    _add_offset(idxb, G2, c * HID)

    plsc.subcore_barrier()

    _layer(h_hbm, idxb, oidxb, wb, G2, acc2_sh, rows, gsems, ssems)

    plsc.subcore_barrier()

    @pl.when(s < 8)
    def _():
        pltpu.sync_copy(acc2_sh.at[pl.ds(s * 8, 8)], o_hbm.at[c, pl.ds(s * 8, 8)])


def kernel(x, connections1, connections2, w1, b1, w2, b2):
    x2 = x.reshape(NC, HB, IN).transpose(0, 2, 1).reshape(NC * IN, HB)
    in1 = connections1[1].reshape(NNZ1 // CH, CH)
    out1 = connections1[0].reshape(NNZ1 // CH, CH)
    in2 = connections2[1].reshape(NNZ2 // CH, CH)
    out2 = connections2[0].reshape(NNZ2 // CH, CH)
    w1r = w1.reshape(NNZ1 // CH, CH)
    w2r = w2.reshape(NNZ2 // CH, CH)

    mesh = plsc.VectorSubcoreMesh(core_axis_name="c", subcore_axis_name="s")
    h_hbm, o_hbm = pl.kernel(
        _sc_body,
        out_type=(
            jax.ShapeDtypeStruct((NC * HID, HB), jnp.float32),
            jax.ShapeDtypeStruct((NC, OUT, HB), jnp.float32),
        ),
        mesh=mesh,
        scratch_types=(
            pltpu.VMEM_SHARED((HID, HB), jnp.float32),
            pltpu.VMEM_SHARED((OUT, HB), jnp.float32),
            pltpu.VMEM((G1, CH), jnp.int32),
            pltpu.VMEM((G1, CH), jnp.int32),
            pltpu.VMEM((G1, CH), jnp.float32),
            pltpu.VMEM((CH, HB), jnp.float32),
            pltpu.VMEM((CH, HB), jnp.float32),
            pltpu.VMEM((CH, HB), jnp.float32),
            pltpu.VMEM((CH, HB), jnp.float32),
            pltpu.VMEM((R1 + 16,), jnp.float32),
            pltpu.VMEM((24,), jnp.float32),
            pltpu.SemaphoreType.DMA,
            pltpu.SemaphoreType.DMA,
            pltpu.SemaphoreType.DMA,
            pltpu.SemaphoreType.DMA,
            pltpu.SemaphoreType.DMA,
            pltpu.SemaphoreType.DMA,
            pltpu.SemaphoreType.DMA,
            pltpu.SemaphoreType.DMA,
        ),
    )(x2, in1, out1, w1r, b1, in2, out2, w2r, b2)
    del h_hbm
    return o_hbm.transpose(0, 2, 1).reshape(B, OUT)
